# baseline (speedup 1.0000x reference)
import functools

import jax
import jax.numpy as jnp
from jax import lax
from jax.experimental import pallas as pl
from jax.experimental.pallas import tpu as pltpu
from jax.experimental.pallas import tpu_sc as plsc

_R = 5
_N = 4096
_L = 4096
_NC = 2
_NS = 16
_NW = _NC * _NS
_ROWS = _N // _NW
_LANES = 16

_NEGINF = float("-inf")
_POSINF = float("inf")


def _perm(v, idx):
    return lax.gather(
        v, idx[:, None],
        lax.GatherDimensionNumbers(offset_dims=(), collapsed_slice_dims=(0,),
                                   start_index_map=(0,)),
        slice_sizes=(1,), mode=lax.GatherScatterMode.PROMISE_IN_BOUNDS)


def _row_result(buf, leff):
    iota = lax.iota(jnp.int32, _LANES)
    vec = jnp.full((_LANES,), _POSINF, jnp.float32)
    for j in range(_R):
        s = (j * leff) // _R
        e = ((j + 1) * leff + (_R - 1)) // _R
        c0 = (s // _LANES) * _LANES
        ntrip = (e - c0 + (_LANES - 1)) // _LANES

        def body(t, carry, c0=c0, s=s, e=e):
            am, an = carry
            p0 = c0 + t * _LANES
            v = buf[pl.ds(pl.multiple_of(p0, _LANES), _LANES)]
            pos = p0 + iota
            m = (pos >= s) & (pos < e)
            am = jnp.maximum(am, jnp.where(m, v, _NEGINF))
            an = jnp.minimum(an, jnp.where(m, v, _POSINF))
            return am, an

        am, an = lax.fori_loop(
            0, ntrip, body,
            (jnp.full((_LANES,), _NEGINF, jnp.float32),
             jnp.full((_LANES,), _POSINF, jnp.float32)))
        for sh in (1, 2, 4, 8):
            am = jnp.maximum(am, _perm(am, iota ^ sh))
            an = jnp.minimum(an, _perm(an, iota ^ sh))
        vec = jnp.where(iota == j, an, vec)
        vec = jnp.where(iota == (_R + j), am, vec)
    for k in (2, 4, 8, 16):
        sh = k // 2
        while sh >= 1:
            p = _perm(vec, iota ^ sh)
            want_min = ((iota & sh) == 0) != ((iota & k) != 0)
            vec = jnp.where(want_min, jnp.minimum(vec, p),
                            jnp.maximum(vec, p))
            sh //= 2
    return vec


def _sc_body(x_hbm, len_hbm, out_hbm, len_v, buf0, buf1, outv, sem0, sem1):
    wid = lax.axis_index("s") * _NC + lax.axis_index("c")
    base = wid * _ROWS
    pltpu.sync_copy(len_hbm.at[pl.ds(base * _LANES, _ROWS * _LANES)], len_v)
    bufs = (buf0, buf1)
    sems = (sem0, sem1)
    pltpu.async_copy(x_hbm.at[base], buf0, sem0)

    def outer(i2, _):
        for k in range(2):
            i = i2 * 2 + k
            cur = bufs[k]
            nxt = bufs[1 - k]

            @pl.when(i + 1 < _ROWS)
            def _():
                pltpu.async_copy(x_hbm.at[base + i + 1], nxt, sems[1 - k])

            pltpu.make_async_copy(x_hbm.at[base + i], cur, sems[k]).wait()
            lv = len_v[pl.ds(pl.multiple_of(i * _LANES, _LANES), _LANES)]
            leff = lv[0]
            outv[i, :] = _row_result(cur, leff)
        return 0

    lax.fori_loop(0, _ROWS // 2, outer, 0)
    pltpu.sync_copy(outv, out_hbm.at[pl.ds(base, _ROWS)])


@jax.jit
def _minmax16(inputs, lengths16):
    mesh = plsc.VectorSubcoreMesh(core_axis_name="c", subcore_axis_name="s")
    f = functools.partial(
        pl.kernel,
        out_type=jax.ShapeDtypeStruct((_N, _LANES), jnp.float32),
        mesh=mesh,
        scratch_types=[
            pltpu.VMEM((_ROWS * _LANES,), jnp.int32),
            pltpu.VMEM((_L,), jnp.float32),
            pltpu.VMEM((_L,), jnp.float32),
            pltpu.VMEM((_ROWS, _LANES), jnp.float32),
            pltpu.SemaphoreType.DMA,
            pltpu.SemaphoreType.DMA,
        ],
    )(_sc_body)
    return f(inputs, lengths16)


def kernel(inputs, lengths):
    lengths16 = jnp.repeat(jnp.clip(lengths.astype(jnp.int32), 1, _L), _LANES)
    out16 = _minmax16(inputs, lengths16)
    return out16[:, : 2 * _R]
